# Initial kernel scaffold; baseline (speedup 1.0000x reference)
import jax
import jax.numpy as jnp
from jax.experimental import pallas as pl


def kernel(x, edge_index_all, W0, b0, W1, b1):
    raise NotImplementedError("write your pallas kernel here")


# speedup vs baseline: 16.3985x; 16.3985x over previous
import functools

import jax
import jax.numpy as jnp
from jax import lax
from jax.experimental import pallas as pl
from jax.experimental.pallas import tpu as pltpu
from jax.experimental.pallas import tpu_sc as plsc

N = 10000
NPAD = 10240
D = 128
NCORE = 2
SUB = 16
NW = NCORE * SUB
CHUNK = 80
ROWS_PER_SUB = NPAD // SUB
BR = 1024


def _sc_mesh():
    return plsc.VectorSubcoreMesh(core_axis_name="c", subcore_axis_name="s")


def _deg(dst3, ones1, zeros1):
    nchunk = dst3.shape[1]

    @functools.partial(
        pl.kernel,
        out_type=jax.ShapeDtypeStruct((NCORE * NPAD,), jnp.float32),
        mesh=_sc_mesh(),
        scratch_types=[
            pltpu.VMEM((nchunk, CHUNK), jnp.int32),
            pltpu.VMEM((CHUNK,), jnp.float32),
            pltpu.VMEM_SHARED((NPAD,), jnp.float32),
        ],
    )
    def deg_kernel(dst_hbm, ones_hbm, z_hbm, out_hbm, dst_v, ones_v, acc):
        cid = lax.axis_index("c")
        sid = lax.axis_index("s")
        wid = cid * SUB + sid
        pltpu.sync_copy(dst_hbm.at[wid], dst_v)
        pltpu.sync_copy(ones_hbm, ones_v)
        pltpu.sync_copy(z_hbm, acc.at[pl.ds(sid * ROWS_PER_SUB, ROWS_PER_SUB)])
        plsc.subcore_barrier()

        @pl.loop(0, nchunk)
        def _(j):
            pltpu.sync_copy(ones_v, acc.at[dst_v.at[j]], add=True)

        plsc.subcore_barrier()
        pltpu.sync_copy(
            acc.at[pl.ds(sid * ROWS_PER_SUB, ROWS_PER_SUB)],
            out_hbm.at[pl.ds(cid * NPAD + sid * ROWS_PER_SUB, ROWS_PER_SUB)],
        )

    return deg_kernel(dst3, ones1, zeros1)


def _agg(g, src3, dst3, zeros128):
    nchunk = src3.shape[1]

    @functools.partial(
        pl.kernel,
        out_type=jax.ShapeDtypeStruct((NCORE * NPAD, D), jnp.float32),
        mesh=_sc_mesh(),
        scratch_types=[
            pltpu.VMEM((nchunk, CHUNK), jnp.int32),
            pltpu.VMEM((nchunk, CHUNK), jnp.int32),
            pltpu.VMEM((CHUNK, D), jnp.float32),
            pltpu.VMEM_SHARED((NPAD, D), jnp.float32),
        ],
    )
    def agg_kernel(g_hbm, src_hbm, dst_hbm, z_hbm, out_hbm, src_v, dst_v, rows_v, acc):
        cid = lax.axis_index("c")
        sid = lax.axis_index("s")
        wid = cid * SUB + sid
        pltpu.sync_copy(src_hbm.at[wid], src_v)
        pltpu.sync_copy(dst_hbm.at[wid], dst_v)
        pltpu.sync_copy(z_hbm, acc.at[pl.ds(sid * ROWS_PER_SUB, ROWS_PER_SUB)])
        plsc.subcore_barrier()

        @pl.loop(0, nchunk)
        def _(j):
            pltpu.sync_copy(g_hbm.at[src_v.at[j]], rows_v)
            pltpu.sync_copy(rows_v, acc.at[dst_v.at[j]], add=True)

        plsc.subcore_barrier()
        pltpu.sync_copy(
            acc.at[pl.ds(sid * ROWS_PER_SUB, ROWS_PER_SUB)],
            out_hbm.at[pl.ds(cid * NPAD + sid * ROWS_PER_SUB, ROWS_PER_SUB)],
        )

    return agg_kernel(g, src3, dst3, zeros128)


def _dinv_block(dp_ref):
    deg = dp_ref[0] + dp_ref[1]
    return lax.rsqrt(jnp.maximum(deg, 1.0))


def _tc_prep(degp, x, w0):
    def body(dp_ref, x_ref, w_ref, o_ref):
        d = _dinv_block(dp_ref)
        ht = jnp.dot(x_ref[...], w_ref[...], preferred_element_type=jnp.float32)
        o_ref[...] = ht * d

    return pl.pallas_call(
        body,
        grid=(NPAD // BR,),
        in_specs=[
            pl.BlockSpec((2, BR, 1), lambda i: (0, i, 0)),
            pl.BlockSpec((BR, D), lambda i: (i, 0)),
            pl.BlockSpec((D, D), lambda i: (0, 0)),
        ],
        out_specs=pl.BlockSpec((BR, D), lambda i: (i, 0)),
        out_shape=jax.ShapeDtypeStruct((NPAD, D), jnp.float32),
    )(degp, x, w0)


def _tc_mid(p2, degp, b0, w1):
    def body(p_ref, dp_ref, b_ref, w_ref, o_ref):
        d = _dinv_block(dp_ref)
        s = p_ref[0] + p_ref[1]
        h = jnp.tanh(s * d + b_ref[...])
        o_ref[...] = jnp.dot(h, w_ref[...], preferred_element_type=jnp.float32) * d

    return pl.pallas_call(
        body,
        grid=(NPAD // BR,),
        in_specs=[
            pl.BlockSpec((2, BR, D), lambda i: (0, i, 0)),
            pl.BlockSpec((2, BR, 1), lambda i: (0, i, 0)),
            pl.BlockSpec((1, D), lambda i: (0, 0)),
            pl.BlockSpec((D, D), lambda i: (0, 0)),
        ],
        out_specs=pl.BlockSpec((BR, D), lambda i: (i, 0)),
        out_shape=jax.ShapeDtypeStruct((NPAD, D), jnp.float32),
    )(p2, degp, b0, w1)


def _tc_fin(p2, degp, b1):
    def body(p_ref, dp_ref, b_ref, o_ref):
        d = _dinv_block(dp_ref)
        s = p_ref[0] + p_ref[1]
        o_ref[...] = jnp.tanh(s * d + b_ref[...])

    return pl.pallas_call(
        body,
        grid=(NPAD // BR,),
        in_specs=[
            pl.BlockSpec((2, BR, D), lambda i: (0, i, 0)),
            pl.BlockSpec((2, BR, 1), lambda i: (0, i, 0)),
            pl.BlockSpec((1, D), lambda i: (0, 0)),
        ],
        out_specs=pl.BlockSpec((BR, D), lambda i: (i, 0)),
        out_shape=jax.ShapeDtypeStruct((NPAD, D), jnp.float32),
    )(p2, degp, b1)


def kernel(x, edge_index_all, W0, b0, W1, b1):
    src3 = edge_index_all[0].reshape(NW, -1, CHUNK)
    dst3 = edge_index_all[1].reshape(NW, -1, CHUNK)
    zeros128 = jnp.zeros((ROWS_PER_SUB, D), jnp.float32)
    zeros1 = jnp.zeros((ROWS_PER_SUB,), jnp.float32)
    ones1 = jnp.ones((CHUNK,), jnp.float32)
    xpad = jnp.pad(x, ((0, NPAD - N), (0, 0)))

    degp = _deg(dst3, ones1, zeros1).reshape(NCORE, NPAD, 1)
    g0 = _tc_prep(degp, xpad, W0)
    p1 = _agg(g0, src3, dst3, zeros128).reshape(NCORE, NPAD, D)
    g1 = _tc_mid(p1, degp, b0.reshape(1, D), W1)
    p2 = _agg(g1, src3, dst3, zeros128).reshape(NCORE, NPAD, D)
    out = _tc_fin(p2, degp, b1.reshape(1, D))
    return out[:N]


# speedup vs baseline: 23.7098x; 1.4459x over previous
import functools

import jax
import jax.numpy as jnp
from jax import lax
from jax.experimental import pallas as pl
from jax.experimental.pallas import tpu as pltpu
from jax.experimental.pallas import tpu_sc as plsc

N = 10000
NPAD = 10240
D = 128
NCORE = 2
SUB = 16
NW = NCORE * SUB
CHUNK = 125
ROWS_PER_SUB = NPAD // SUB
BR = 1024


def _sc_mesh():
    return plsc.VectorSubcoreMesh(core_axis_name="c", subcore_axis_name="s")


def _deg(dst3, ones1, zeros1):
    nchunk = dst3.shape[1]

    @functools.partial(
        pl.kernel,
        out_type=jax.ShapeDtypeStruct((NCORE * NPAD,), jnp.float32),
        mesh=_sc_mesh(),
        scratch_types=[
            pltpu.VMEM((nchunk, CHUNK), jnp.int32),
            pltpu.VMEM((CHUNK,), jnp.float32),
            pltpu.VMEM_SHARED((NPAD,), jnp.float32),
        ],
    )
    def deg_kernel(dst_hbm, ones_hbm, z_hbm, out_hbm, dst_v, ones_v, acc):
        cid = lax.axis_index("c")
        sid = lax.axis_index("s")
        wid = cid * SUB + sid
        pltpu.sync_copy(dst_hbm.at[wid], dst_v)
        pltpu.sync_copy(ones_hbm, ones_v)
        pltpu.sync_copy(z_hbm, acc.at[pl.ds(sid * ROWS_PER_SUB, ROWS_PER_SUB)])
        plsc.subcore_barrier()

        @pl.loop(0, nchunk)
        def _(j):
            pltpu.sync_copy(ones_v, acc.at[dst_v.at[j]], add=True)

        plsc.subcore_barrier()
        pltpu.sync_copy(
            acc.at[pl.ds(sid * ROWS_PER_SUB, ROWS_PER_SUB)],
            out_hbm.at[pl.ds(cid * NPAD + sid * ROWS_PER_SUB, ROWS_PER_SUB)],
        )

    return deg_kernel(dst3, ones1, zeros1)


def _agg(g, src2, dst2, zeros128):
    nchunk = src2.shape[0] // NW

    @functools.partial(
        pl.kernel,
        out_type=jax.ShapeDtypeStruct((NCORE * NPAD, D), jnp.float32),
        mesh=_sc_mesh(),
        scratch_types=[
            pltpu.VMEM((CHUNK,), jnp.int32),
            pltpu.VMEM((CHUNK,), jnp.int32),
            pltpu.VMEM((CHUNK,), jnp.int32),
            pltpu.VMEM((CHUNK,), jnp.int32),
            pltpu.VMEM((CHUNK, D), jnp.float32),
            pltpu.VMEM((CHUNK, D), jnp.float32),
            pltpu.VMEM_SHARED((NPAD, D), jnp.float32),
            pltpu.SemaphoreType.DMA,
            pltpu.SemaphoreType.DMA,
            pltpu.SemaphoreType.DMA,
            pltpu.SemaphoreType.DMA,
            pltpu.SemaphoreType.DMA,
            pltpu.SemaphoreType.DMA,
        ],
    )
    def agg_kernel(g_hbm, src_hbm, dst_hbm, z_hbm, out_hbm,
                   s0, s1, d0, d1, r0, r1, acc,
                   ss0, ss1, sd0, sd1, sg0, sg1):
        cid = lax.axis_index("c")
        sid = lax.axis_index("s")
        wid = cid * SUB + sid
        base = wid * nchunk
        sbuf, dbuf, rbuf = (s0, s1), (d0, d1), (r0, r1)
        ssem, dsem, gsem = (ss0, ss1), (sd0, sd1), (sg0, sg1)

        pltpu.sync_copy(z_hbm, acc.at[pl.ds(sid * ROWS_PER_SUB, ROWS_PER_SUB)])
        plsc.subcore_barrier()

        pltpu.async_copy(src_hbm.at[base], s0, ss0)
        pltpu.async_copy(dst_hbm.at[base], d0, sd0)
        pltpu.make_async_copy(src_hbm.at[base], s0, ss0).wait()
        pltpu.async_copy(g_hbm.at[s0], r0, sg0)
        pltpu.async_copy(src_hbm.at[base + 1], s1, ss1)

        @pl.loop(0, nchunk, step=2)
        def _(j):
            for b in range(2):
                jj = j + b
                o = 1 - b
                pltpu.make_async_copy(g_hbm.at[sbuf[b]], rbuf[b], gsem[b]).wait()

                @pl.when(jj + 1 < nchunk)
                def _():
                    pltpu.make_async_copy(src_hbm.at[base + jj + 1], sbuf[o], ssem[o]).wait()
                    pltpu.async_copy(g_hbm.at[sbuf[o]], rbuf[o], gsem[o])
                    pltpu.async_copy(dst_hbm.at[base + jj + 1], dbuf[o], dsem[o])

                @pl.when(jj + 2 < nchunk)
                def _():
                    pltpu.async_copy(src_hbm.at[base + jj + 2], sbuf[b], ssem[b])

                pltpu.make_async_copy(dst_hbm.at[base + jj], dbuf[b], dsem[b]).wait()
                pltpu.sync_copy(rbuf[b], acc.at[dbuf[b]], add=True)

        plsc.subcore_barrier()
        pltpu.sync_copy(
            acc.at[pl.ds(sid * ROWS_PER_SUB, ROWS_PER_SUB)],
            out_hbm.at[pl.ds(cid * NPAD + sid * ROWS_PER_SUB, ROWS_PER_SUB)],
        )

    return agg_kernel(g, src2, dst2, zeros128)


def _dinv_block(dp_ref):
    deg = dp_ref[0] + dp_ref[1]
    return lax.rsqrt(jnp.maximum(deg, 1.0))


def _tc_prep(degp, x, w0):
    def body(dp_ref, x_ref, w_ref, o_ref):
        d = _dinv_block(dp_ref)
        ht = jnp.dot(x_ref[...], w_ref[...], preferred_element_type=jnp.float32)
        o_ref[...] = ht * d

    return pl.pallas_call(
        body,
        grid=(NPAD // BR,),
        in_specs=[
            pl.BlockSpec((2, BR, 1), lambda i: (0, i, 0)),
            pl.BlockSpec((BR, D), lambda i: (i, 0)),
            pl.BlockSpec((D, D), lambda i: (0, 0)),
        ],
        out_specs=pl.BlockSpec((BR, D), lambda i: (i, 0)),
        out_shape=jax.ShapeDtypeStruct((NPAD, D), jnp.float32),
    )(degp, x, w0)


def _tc_mid(p2, degp, b0, w1):
    def body(p_ref, dp_ref, b_ref, w_ref, o_ref):
        d = _dinv_block(dp_ref)
        s = p_ref[0] + p_ref[1]
        h = jnp.tanh(s * d + b_ref[...])
        o_ref[...] = jnp.dot(h, w_ref[...], preferred_element_type=jnp.float32) * d

    return pl.pallas_call(
        body,
        grid=(NPAD // BR,),
        in_specs=[
            pl.BlockSpec((2, BR, D), lambda i: (0, i, 0)),
            pl.BlockSpec((2, BR, 1), lambda i: (0, i, 0)),
            pl.BlockSpec((1, D), lambda i: (0, 0)),
            pl.BlockSpec((D, D), lambda i: (0, 0)),
        ],
        out_specs=pl.BlockSpec((BR, D), lambda i: (i, 0)),
        out_shape=jax.ShapeDtypeStruct((NPAD, D), jnp.float32),
    )(p2, degp, b0, w1)


def _tc_fin(p2, degp, b1):
    def body(p_ref, dp_ref, b_ref, o_ref):
        d = _dinv_block(dp_ref)
        s = p_ref[0] + p_ref[1]
        o_ref[...] = jnp.tanh(s * d + b_ref[...])

    return pl.pallas_call(
        body,
        grid=(NPAD // BR,),
        in_specs=[
            pl.BlockSpec((2, BR, D), lambda i: (0, i, 0)),
            pl.BlockSpec((2, BR, 1), lambda i: (0, i, 0)),
            pl.BlockSpec((1, D), lambda i: (0, 0)),
        ],
        out_specs=pl.BlockSpec((BR, D), lambda i: (i, 0)),
        out_shape=jax.ShapeDtypeStruct((NPAD, D), jnp.float32),
    )(p2, degp, b1)


def kernel(x, edge_index_all, W0, b0, W1, b1):
    src2 = edge_index_all[0].reshape(-1, CHUNK)
    dst2 = edge_index_all[1].reshape(-1, CHUNK)
    dst3 = edge_index_all[1].reshape(NW, -1, CHUNK)
    zeros128 = jnp.zeros((ROWS_PER_SUB, D), jnp.float32)
    zeros1 = jnp.zeros((ROWS_PER_SUB,), jnp.float32)
    ones1 = jnp.ones((CHUNK,), jnp.float32)
    xpad = jnp.pad(x, ((0, NPAD - N), (0, 0)))

    degp = _deg(dst3, ones1, zeros1).reshape(NCORE, NPAD, 1)
    g0 = _tc_prep(degp, xpad, W0)
    p1 = _agg(g0, src2, dst2, zeros128).reshape(NCORE, NPAD, D)
    g1 = _tc_mid(p1, degp, b0.reshape(1, D), W1)
    p2 = _agg(g1, src2, dst2, zeros128).reshape(NCORE, NPAD, D)
    out = _tc_fin(p2, degp, b1.reshape(1, D))
    return out[:N]


# speedup vs baseline: 27.0173x; 1.1395x over previous
import functools

import jax
import jax.numpy as jnp
from jax import lax
from jax.experimental import pallas as pl
from jax.experimental.pallas import tpu as pltpu
from jax.experimental.pallas import tpu_sc as plsc

N = 10000
NPAD = 10240
D = 128
NCORE = 2
SUB = 16
NW = NCORE * SUB
CHUNK = 125
ROWS_PER_SUB = NPAD // SUB
BR = 1024


def _sc_mesh():
    return plsc.VectorSubcoreMesh(core_axis_name="c", subcore_axis_name="s")


def _deg(dst3, ones1, zeros1):
    nchunk = dst3.shape[1]

    @functools.partial(
        pl.kernel,
        out_type=jax.ShapeDtypeStruct((NCORE * NPAD,), jnp.float32),
        mesh=_sc_mesh(),
        scratch_types=[
            pltpu.VMEM((nchunk, CHUNK), jnp.int32),
            pltpu.VMEM((CHUNK,), jnp.float32),
            pltpu.VMEM_SHARED((NPAD,), jnp.float32),
        ],
    )
    def deg_kernel(dst_hbm, ones_hbm, z_hbm, out_hbm, dst_v, ones_v, acc):
        cid = lax.axis_index("c")
        sid = lax.axis_index("s")
        wid = cid * SUB + sid
        pltpu.sync_copy(dst_hbm.at[wid], dst_v)
        pltpu.sync_copy(ones_hbm, ones_v)
        pltpu.sync_copy(z_hbm, acc.at[pl.ds(sid * ROWS_PER_SUB, ROWS_PER_SUB)])
        plsc.subcore_barrier()

        @pl.loop(0, nchunk)
        def _(j):
            pltpu.sync_copy(ones_v, acc.at[dst_v.at[j]], add=True)

        plsc.subcore_barrier()
        pltpu.sync_copy(
            acc.at[pl.ds(sid * ROWS_PER_SUB, ROWS_PER_SUB)],
            out_hbm.at[pl.ds(cid * NPAD + sid * ROWS_PER_SUB, ROWS_PER_SUB)],
        )

    return deg_kernel(dst3, ones1, zeros1)


def _agg(g, src2, dst2, zeros128):
    nchunk = src2.shape[0] // NW

    @functools.partial(
        pl.kernel,
        out_type=jax.ShapeDtypeStruct((NCORE * NPAD, D), jnp.float32),
        mesh=_sc_mesh(),
        scratch_types=[
            pltpu.VMEM((CHUNK,), jnp.int32),
            pltpu.VMEM((CHUNK,), jnp.int32),
            pltpu.VMEM((CHUNK,), jnp.int32),
            pltpu.VMEM((CHUNK,), jnp.int32),
            pltpu.VMEM((CHUNK, D), jnp.float32),
            pltpu.VMEM((CHUNK, D), jnp.float32),
            pltpu.VMEM_SHARED((NPAD, D), jnp.float32),
            pltpu.SemaphoreType.DMA,
            pltpu.SemaphoreType.DMA,
            pltpu.SemaphoreType.DMA,
            pltpu.SemaphoreType.DMA,
            pltpu.SemaphoreType.DMA,
            pltpu.SemaphoreType.DMA,
            pltpu.SemaphoreType.DMA,
            pltpu.SemaphoreType.DMA,
        ],
    )
    def agg_kernel(g_hbm, src_hbm, dst_hbm, z_hbm, out_hbm,
                   s0, s1, d0, d1, r0, r1, acc,
                   ss0, ss1, sd0, sd1, sg0, sg1, sc0, sc1):
        cid = lax.axis_index("c")
        sid = lax.axis_index("s")
        wid = cid * SUB + sid
        base = wid * nchunk
        sbuf, dbuf, rbuf = (s0, s1), (d0, d1), (r0, r1)
        ssem, dsem, gsem, csem = (ss0, ss1), (sd0, sd1), (sg0, sg1), (sc0, sc1)

        pltpu.sync_copy(z_hbm, acc.at[pl.ds(sid * ROWS_PER_SUB, ROWS_PER_SUB)])
        plsc.subcore_barrier()

        pltpu.async_copy(src_hbm.at[base], s0, ss0)
        pltpu.async_copy(dst_hbm.at[base], d0, sd0)
        pltpu.make_async_copy(src_hbm.at[base], s0, ss0).wait()
        pltpu.async_copy(g_hbm.at[s0], r0, sg0)
        pltpu.async_copy(src_hbm.at[base + 1], s1, ss1)

        @pl.loop(0, nchunk, step=2)
        def _(j):
            for b in range(2):
                jj = j + b
                o = 1 - b

                @pl.when(jj >= 1)
                def _():
                    pltpu.make_async_copy(rbuf[o], acc.at[dbuf[o]], csem[o]).wait()

                @pl.when(jj + 1 < nchunk)
                def _():
                    pltpu.make_async_copy(src_hbm.at[base + jj + 1], sbuf[o], ssem[o]).wait()
                    pltpu.async_copy(g_hbm.at[sbuf[o]], rbuf[o], gsem[o])
                    pltpu.async_copy(dst_hbm.at[base + jj + 1], dbuf[o], dsem[o])

                pltpu.make_async_copy(g_hbm.at[sbuf[b]], rbuf[b], gsem[b]).wait()

                @pl.when(jj + 2 < nchunk)
                def _():
                    pltpu.async_copy(src_hbm.at[base + jj + 2], sbuf[b], ssem[b])

                pltpu.make_async_copy(dst_hbm.at[base + jj], dbuf[b], dsem[b]).wait()
                pltpu.async_copy(rbuf[b], acc.at[dbuf[b]], csem[b], add=True)

        pltpu.make_async_copy(rbuf[(nchunk - 1) % 2], acc.at[dbuf[(nchunk - 1) % 2]],
                              csem[(nchunk - 1) % 2]).wait()
        plsc.subcore_barrier()
        pltpu.sync_copy(
            acc.at[pl.ds(sid * ROWS_PER_SUB, ROWS_PER_SUB)],
            out_hbm.at[pl.ds(cid * NPAD + sid * ROWS_PER_SUB, ROWS_PER_SUB)],
        )

    return agg_kernel(g, src2, dst2, zeros128)


def _dinv_block(dp_ref):
    deg = dp_ref[0] + dp_ref[1]
    return lax.rsqrt(jnp.maximum(deg, 1.0))


def _tc_prep(degp, x, w0):
    def body(dp_ref, x_ref, w_ref, o_ref):
        d = _dinv_block(dp_ref)
        ht = jnp.dot(x_ref[...], w_ref[...], preferred_element_type=jnp.float32)
        o_ref[...] = ht * d

    return pl.pallas_call(
        body,
        grid=(NPAD // BR,),
        in_specs=[
            pl.BlockSpec((2, BR, 1), lambda i: (0, i, 0)),
            pl.BlockSpec((BR, D), lambda i: (i, 0)),
            pl.BlockSpec((D, D), lambda i: (0, 0)),
        ],
        out_specs=pl.BlockSpec((BR, D), lambda i: (i, 0)),
        out_shape=jax.ShapeDtypeStruct((NPAD, D), jnp.float32),
    )(degp, x, w0)


def _tc_mid(p2, degp, b0, w1):
    def body(p_ref, dp_ref, b_ref, w_ref, o_ref):
        d = _dinv_block(dp_ref)
        s = p_ref[0] + p_ref[1]
        h = jnp.tanh(s * d + b_ref[...])
        o_ref[...] = jnp.dot(h, w_ref[...], preferred_element_type=jnp.float32) * d

    return pl.pallas_call(
        body,
        grid=(NPAD // BR,),
        in_specs=[
            pl.BlockSpec((2, BR, D), lambda i: (0, i, 0)),
            pl.BlockSpec((2, BR, 1), lambda i: (0, i, 0)),
            pl.BlockSpec((1, D), lambda i: (0, 0)),
            pl.BlockSpec((D, D), lambda i: (0, 0)),
        ],
        out_specs=pl.BlockSpec((BR, D), lambda i: (i, 0)),
        out_shape=jax.ShapeDtypeStruct((NPAD, D), jnp.float32),
    )(p2, degp, b0, w1)


def _tc_fin(p2, degp, b1):
    def body(p_ref, dp_ref, b_ref, o_ref):
        d = _dinv_block(dp_ref)
        s = p_ref[0] + p_ref[1]
        o_ref[...] = jnp.tanh(s * d + b_ref[...])

    return pl.pallas_call(
        body,
        grid=(NPAD // BR,),
        in_specs=[
            pl.BlockSpec((2, BR, D), lambda i: (0, i, 0)),
            pl.BlockSpec((2, BR, 1), lambda i: (0, i, 0)),
            pl.BlockSpec((1, D), lambda i: (0, 0)),
        ],
        out_specs=pl.BlockSpec((BR, D), lambda i: (i, 0)),
        out_shape=jax.ShapeDtypeStruct((NPAD, D), jnp.float32),
    )(p2, degp, b1)


def kernel(x, edge_index_all, W0, b0, W1, b1):
    src2 = edge_index_all[0].reshape(-1, CHUNK)
    dst2 = edge_index_all[1].reshape(-1, CHUNK)
    dst3 = edge_index_all[1].reshape(NW, -1, CHUNK)
    zeros128 = jnp.zeros((ROWS_PER_SUB, D), jnp.float32)
    zeros1 = jnp.zeros((ROWS_PER_SUB,), jnp.float32)
    ones1 = jnp.ones((CHUNK,), jnp.float32)
    xpad = jnp.pad(x, ((0, NPAD - N), (0, 0)))

    degp = _deg(dst3, ones1, zeros1).reshape(NCORE, NPAD, 1)
    g0 = _tc_prep(degp, xpad, W0)
    p1 = _agg(g0, src2, dst2, zeros128).reshape(NCORE, NPAD, D)
    g1 = _tc_mid(p1, degp, b0.reshape(1, D), W1)
    p2 = _agg(g1, src2, dst2, zeros128).reshape(NCORE, NPAD, D)
    out = _tc_fin(p2, degp, b1.reshape(1, D))
    return out[:N]
